# baseline (speedup 1.0000x reference)
import functools
import jax
import jax.numpy as jnp
from jax.experimental import pallas as pl
from jax.experimental.pallas import tpu as pltpu

_B, _N, _NF = 8, 2048, 4096
_K, _FIN, _FOUT, _FOUT2 = 16, 128, 256, 256
_EPS = 1e-5
_M = _B * _N
_ROWS = 256
_STEPS = _M // _ROWS
_CIN = (1 + _K) * _FIN


def _conv1_body(g_ref, w_ref, b_ref, h_ref, stat_ref, acc_ref):
    i = pl.program_id(0)

    @pl.when(i == 0)
    def _():
        acc_ref[...] = jnp.zeros_like(acc_ref)

    h = jnp.dot(g_ref[...], w_ref[...], preferred_element_type=jnp.float32)
    h = h + b_ref[...]
    h_ref[...] = h
    acc_ref[0, :] += h.sum(axis=0)
    acc_ref[1, :] += (h * h).sum(axis=0)

    @pl.when(i == _STEPS - 1)
    def _():
        stat_ref[...] = acc_ref[...]


def _bn_pool_body(h_ref, sc_ref, sh_ref, p_ref, stat_ref, acc_ref):
    i = pl.program_id(0)

    @pl.when(i == 0)
    def _():
        acc_ref[...] = jnp.zeros_like(acc_ref)

    a = jnp.maximum(h_ref[...] * sc_ref[...] + sh_ref[...], 0.0)
    p = a.reshape(_ROWS, _FOUT // _K, _K).max(axis=-1)
    p_ref[...] = p
    acc_ref[0:16, :] += jnp.dot(p.T, p, preferred_element_type=jnp.float32)
    acc_ref[16, :] += p.sum(axis=0)

    @pl.when(i == _STEPS - 1)
    def _():
        stat_ref[...] = acc_ref[...]


def _conv2_body(p_ref, w_ref, sc_ref, sh_ref, o_ref):
    h = jnp.dot(p_ref[...], w_ref[...], preferred_element_type=jnp.float32)
    o_ref[...] = jnp.maximum(h * sc_ref[...] + sh_ref[...], 0.0)


def _dense_chain(gf, W1r, b1, g1, be1, W2t, b2, g2, be2):
    h1, st1 = pl.pallas_call(
        _conv1_body,
        grid=(_STEPS,),
        in_specs=[
            pl.BlockSpec((_ROWS, _CIN), lambda i: (i, 0)),
            pl.BlockSpec((_CIN, _FOUT), lambda i: (0, 0)),
            pl.BlockSpec((1, _FOUT), lambda i: (0, 0)),
        ],
        out_specs=[
            pl.BlockSpec((_ROWS, _FOUT), lambda i: (i, 0)),
            pl.BlockSpec((2, _FOUT), lambda i: (0, 0)),
        ],
        out_shape=[
            jax.ShapeDtypeStruct((_M, _FOUT), jnp.float32),
            jax.ShapeDtypeStruct((2, _FOUT), jnp.float32),
        ],
        scratch_shapes=[pltpu.VMEM((2, _FOUT), jnp.float32)],
    )(gf, W1r, b1.reshape(1, _FOUT))

    mean1 = st1[0] / _M
    var1 = st1[1] / _M - mean1 * mean1
    sc1 = g1 / jnp.sqrt(var1 + _EPS)
    sh1 = be1 - mean1 * sc1

    pooled, st2 = pl.pallas_call(
        _bn_pool_body,
        grid=(_STEPS,),
        in_specs=[
            pl.BlockSpec((_ROWS, _FOUT), lambda i: (i, 0)),
            pl.BlockSpec((1, _FOUT), lambda i: (0, 0)),
            pl.BlockSpec((1, _FOUT), lambda i: (0, 0)),
        ],
        out_specs=[
            pl.BlockSpec((_ROWS, _FOUT // _K), lambda i: (i, 0)),
            pl.BlockSpec((17, _FOUT // _K), lambda i: (0, 0)),
        ],
        out_shape=[
            jax.ShapeDtypeStruct((_M, _FOUT // _K), jnp.float32),
            jax.ShapeDtypeStruct((17, _FOUT // _K), jnp.float32),
        ],
        scratch_shapes=[pltpu.VMEM((17, _FOUT // _K), jnp.float32)],
    )(h1, sc1.reshape(1, _FOUT), sh1.reshape(1, _FOUT))

    S = st2[0:16, :]
    sp = st2[16, :]
    meanp = sp / _M
    cov = S / _M - meanp[:, None] * meanp[None, :]
    mean2 = W2t.T @ meanp + b2
    var2 = jnp.einsum("co,cd,do->o", W2t, cov, W2t)
    sc2 = g2 / jnp.sqrt(var2 + _EPS)
    sh2 = be2 - mean2 * sc2

    out = pl.pallas_call(
        _conv2_body,
        grid=(_STEPS,),
        in_specs=[
            pl.BlockSpec((_ROWS, _FOUT // _K), lambda i: (i, 0)),
            pl.BlockSpec((_FOUT // _K, _FOUT2), lambda i: (0, 0)),
            pl.BlockSpec((1, _FOUT2), lambda i: (0, 0)),
            pl.BlockSpec((1, _FOUT2), lambda i: (0, 0)),
        ],
        out_specs=pl.BlockSpec((_ROWS, _FOUT2), lambda i: (i, 0)),
        out_shape=jax.ShapeDtypeStruct((_M, _FOUT2), jnp.float32),
    )(pooled, W2t, sc2.reshape(1, _FOUT2), sh2.reshape(1, _FOUT2))
    return out.reshape(_B, _N, _FOUT2)


def _topk_neighbors_xla(faces):
    s = jnp.sort(faces, axis=-1)
    minv, midv, maxv = s[..., 0], s[..., 1], s[..., 2]
    valid = jnp.cumprod((minv != midv).astype(jnp.float32), axis=-1)

    def one(mn, md, mx, va):
        adj = jnp.zeros((_N, _N), dtype=jnp.float32)
        adj = adj.at[mn, md].max(va)
        adj = adj.at[mn, mx].max(va)
        adj = adj.at[md, mx].max(va)
        adj = adj + adj.T
        return jax.lax.top_k(adj, _K)

    return jax.vmap(one)(minv, midv, maxv, valid)


def kernel(x, faces, W1, b1, g1, be1, W2, b2, g2, be2):
    vals, idx = _topk_neighbors_xla(faces)
    gathered = jax.vmap(lambda xb, ib: xb[ib])(x, idx)
    g = vals[..., None] * gathered
    gf = jnp.concatenate([x[:, :, None, :], g], axis=2).reshape(_M, _CIN)

    W1r = jnp.transpose(W1[:, :, 0, :], (2, 1, 0)).reshape(_CIN, _FOUT)
    W2t = W2[:, :, 0].T
    return _dense_chain(gf, W1r, b1, g1, be1, W2t, b2, g2, be2)


# speedup vs baseline: 2.3500x; 2.3500x over previous
import functools
import jax
import jax.numpy as jnp
from jax import lax
from jax.experimental import pallas as pl
from jax.experimental.pallas import tpu as pltpu
from jax.experimental.pallas import tpu_sc as plsc

_B, _N, _NF = 8, 2048, 4096
_K, _FIN, _FOUT, _FOUT2 = 16, 128, 256, 256
_EPS = 1e-5
_M = _B * _N
_NE = 6 * _NF
_ROWS = 256
_STEPS = _M // _ROWS
_RG = 32


def _sc_topk_gather(edge_keys, x2):
    mesh = plsc.VectorSubcoreMesh(
        core_axis_name="c", subcore_axis_name="s", num_cores=2, num_subcores=16)

    @functools.partial(
        pl.kernel,
        out_type=[
            jax.ShapeDtypeStruct((_M * _K, _FIN), jnp.float32),
            jax.ShapeDtypeStruct((_M * _K,), jnp.float32),
        ],
        mesh=mesh,
        compiler_params=pltpu.CompilerParams(needs_layout_passes=False),
        scratch_types=[
            pltpu.VMEM((_RG * 2048,), jnp.float32),
            pltpu.VMEM((_NE,), jnp.int32),
            pltpu.VMEM((1024,), jnp.int32),
            pltpu.VMEM((512,), jnp.float32),
            pltpu.VMEM((4, 128), jnp.int32),
            pltpu.VMEM((128, _FIN), jnp.float32),
            pltpu.SemaphoreType.DMA,
        ],
    )
    def body(ek_hbm, x_hbm, g_hbm, v_hbm, mat, ekeys, colbuf, valbuf, gidx,
             gbuf, sem):
        c = lax.axis_index("c")
        s = lax.axis_index("s")
        iota16 = lax.iota(jnp.int32, 16)
        zero16 = jnp.zeros((16,), jnp.float32)

        def task(j, carry):
            b = 4 * c + j // 4
            rg = s + 16 * (j % 4)
            r0 = rg * _RG

            @pl.when((j % 4) == 0)
            def _():
                pltpu.sync_copy(ek_hbm.at[pl.ds(b * _NE, _NE)], ekeys)

            def zrow(i, carry):
                for h in range(8):
                    mat[pl.ds(i * 128 + h * 16, 16)] = zero16
                return carry
            lax.fori_loop(0, _RG * 2048 // 128, zrow, 0)

            def sed(i, carry):
                k = ekeys[pl.ds(i * 16, 16)]
                src = k >> 11
                dst = k & 2047
                m = (src >= r0) & (src < r0 + _RG)
                addr = jnp.where(m, (src - r0) * 2048 + dst, 0)
                val = jnp.where(src == dst, 2.0, 1.0).astype(jnp.float32)
                plsc.store_scatter(mat, [addr], val, mask=m)
                return carry
            lax.fori_loop(0, _NE // 16, sed, 0)

            def row(r, carry):
                p = r0 + r
                a0 = r * 2048
                dv = jnp.max(plsc.load_gather(
                    mat, [jnp.full((16,), a0 + p, jnp.int32)]))

                def cond(st):
                    k, cur = st
                    return (k < 128) & (cur < 16)

                def step(st):
                    k, cur = st
                    w = mat[pl.ds(a0 + k * 16, 16)]
                    colv = k * 16 + iota16
                    m = (w != 0.0) & (colv != p)
                    cnt = jnp.sum(m.astype(jnp.int32))

                    @pl.when(cnt > 0)
                    def _():
                        plsc.store_compressed(
                            colbuf.at[pl.ds(r * 32 + cur, 16)], colv, mask=m)
                    return k + 1, cur + cnt

                _, deg = lax.while_loop(cond, step, (0, 0))

                hd = (dv > 0.0).astype(jnp.int32)
                srcp = iota16 - hd
                cidx = plsc.load_gather(colbuf, [r * 32 + jnp.clip(srcp, 0, 31)])
                isd = (iota16 == 0) & (hd == 1)
                one = (srcp >= 0) & (srcp < deg)
                vals = jnp.where(isd, 2.0, jnp.where(one, 1.0, 0.0))
                idxr = jnp.where(isd, p, jnp.where(one, cidx, 0))
                valbuf[pl.ds(r * 16, 16)] = vals
                gidx[r // 8, pl.ds((r % 8) * 16, 16)] = b * 2048 + idxr
                return carry
            lax.fori_loop(0, _RG, row, 0)

            out0 = (b * 2048 + r0) * _K
            for q in range(4):
                pltpu.async_copy(x_hbm.at[gidx.at[q]], gbuf, sem).wait()
                pltpu.sync_copy(gbuf, g_hbm.at[pl.ds(out0 + q * 128, 128)])
            pltpu.sync_copy(valbuf, v_hbm.at[pl.ds(out0, 512)])
            return carry

        lax.fori_loop(0, 16, task, 0)

    return body(edge_keys, x2)


def _conv1_body(x_ref, g_ref, v_ref, wa_ref, wb_ref, b_ref,
                h_ref, stat_ref, acc_ref):
    i = pl.program_id(0)

    @pl.when(i == 0)
    def _():
        acc_ref[...] = jnp.zeros_like(acc_ref)

    w = g_ref[...] * v_ref[...]
    h = jnp.dot(x_ref[...], wa_ref[...], preferred_element_type=jnp.float32)
    for t in range(_K):
        h += jnp.dot(w[:, t, :], wb_ref[t], preferred_element_type=jnp.float32)
    h = h + b_ref[...]
    h_ref[...] = h
    acc_ref[0, :] += h.sum(axis=0)
    acc_ref[1, :] += (h * h).sum(axis=0)

    @pl.when(i == _STEPS - 1)
    def _():
        stat_ref[...] = acc_ref[...]


def _bn_pool_body(h_ref, sc_ref, sh_ref, p_ref, stat_ref, acc_ref):
    i = pl.program_id(0)

    @pl.when(i == 0)
    def _():
        acc_ref[...] = jnp.zeros_like(acc_ref)

    a = jnp.maximum(h_ref[...] * sc_ref[...] + sh_ref[...], 0.0)
    p = a.reshape(_ROWS, _FOUT // _K, _K).max(axis=-1)
    p_ref[...] = p
    acc_ref[0:16, :] += jnp.dot(p.T, p, preferred_element_type=jnp.float32)
    acc_ref[16, :] += p.sum(axis=0)

    @pl.when(i == _STEPS - 1)
    def _():
        stat_ref[...] = acc_ref[...]


def _conv2_body(p_ref, w_ref, sc_ref, sh_ref, o_ref):
    h = jnp.dot(p_ref[...], w_ref[...], preferred_element_type=jnp.float32)
    o_ref[...] = jnp.maximum(h * sc_ref[...] + sh_ref[...], 0.0)


def _dense_chain(x2, g3, v3, W1a, W1b, b1, g1, be1, W2t, b2, g2, be2):
    h1, st1 = pl.pallas_call(
        _conv1_body,
        grid=(_STEPS,),
        in_specs=[
            pl.BlockSpec((_ROWS, _FIN), lambda i: (i, 0)),
            pl.BlockSpec((_ROWS, _K, _FIN), lambda i: (i, 0, 0)),
            pl.BlockSpec((_ROWS, _K, 1), lambda i: (i, 0, 0)),
            pl.BlockSpec((_FIN, _FOUT), lambda i: (0, 0)),
            pl.BlockSpec((_K, _FIN, _FOUT), lambda i: (0, 0, 0)),
            pl.BlockSpec((1, _FOUT), lambda i: (0, 0)),
        ],
        out_specs=[
            pl.BlockSpec((_ROWS, _FOUT), lambda i: (i, 0)),
            pl.BlockSpec((2, _FOUT), lambda i: (0, 0)),
        ],
        out_shape=[
            jax.ShapeDtypeStruct((_M, _FOUT), jnp.float32),
            jax.ShapeDtypeStruct((2, _FOUT), jnp.float32),
        ],
        scratch_shapes=[pltpu.VMEM((2, _FOUT), jnp.float32)],
    )(x2, g3, v3, W1a, W1b, b1.reshape(1, _FOUT))

    mean1 = st1[0] / _M
    var1 = st1[1] / _M - mean1 * mean1
    sc1 = g1 / jnp.sqrt(var1 + _EPS)
    sh1 = be1 - mean1 * sc1

    pooled, st2 = pl.pallas_call(
        _bn_pool_body,
        grid=(_STEPS,),
        in_specs=[
            pl.BlockSpec((_ROWS, _FOUT), lambda i: (i, 0)),
            pl.BlockSpec((1, _FOUT), lambda i: (0, 0)),
            pl.BlockSpec((1, _FOUT), lambda i: (0, 0)),
        ],
        out_specs=[
            pl.BlockSpec((_ROWS, _FOUT // _K), lambda i: (i, 0)),
            pl.BlockSpec((17, _FOUT // _K), lambda i: (0, 0)),
        ],
        out_shape=[
            jax.ShapeDtypeStruct((_M, _FOUT // _K), jnp.float32),
            jax.ShapeDtypeStruct((17, _FOUT // _K), jnp.float32),
        ],
        scratch_shapes=[pltpu.VMEM((17, _FOUT // _K), jnp.float32)],
    )(h1, sc1.reshape(1, _FOUT), sh1.reshape(1, _FOUT))

    S = st2[0:16, :]
    meanp = st2[16, :] / _M
    cov = S / _M - meanp[:, None] * meanp[None, :]
    mean2 = W2t.T @ meanp + b2
    var2 = jnp.einsum("co,cd,do->o", W2t, cov, W2t)
    sc2 = g2 / jnp.sqrt(var2 + _EPS)
    sh2 = be2 - mean2 * sc2

    out = pl.pallas_call(
        _conv2_body,
        grid=(_STEPS,),
        in_specs=[
            pl.BlockSpec((_ROWS, _FOUT // _K), lambda i: (i, 0)),
            pl.BlockSpec((_FOUT // _K, _FOUT2), lambda i: (0, 0)),
            pl.BlockSpec((1, _FOUT2), lambda i: (0, 0)),
            pl.BlockSpec((1, _FOUT2), lambda i: (0, 0)),
        ],
        out_specs=pl.BlockSpec((_ROWS, _FOUT2), lambda i: (i, 0)),
        out_shape=jax.ShapeDtypeStruct((_M, _FOUT2), jnp.float32),
    )(pooled, W2t, sc2.reshape(1, _FOUT2), sh2.reshape(1, _FOUT2))
    return out.reshape(_B, _N, _FOUT2)


def _edge_keys(faces):
    f0, f1, f2 = faces[..., 0], faces[..., 1], faces[..., 2]
    mn = jnp.minimum(jnp.minimum(f0, f1), f2)
    mx = jnp.maximum(jnp.maximum(f0, f1), f2)
    md = f0 + f1 + f2 - mn - mx
    fi = jnp.arange(_NF, dtype=jnp.int32)
    first_bad = jnp.min(jnp.where(mn == md, fi, _NF), axis=-1)
    valid = fi[None, :] < first_bad[:, None]

    def enc(a, d):
        return jnp.where(valid, a * 2048 + d, -1)

    keys = jnp.stack(
        [enc(mn, md), enc(md, mn), enc(mn, mx),
         enc(mx, mn), enc(md, mx), enc(mx, md)], axis=1)
    return keys.reshape(_B * _NE).astype(jnp.int32)


def kernel(x, faces, W1, b1, g1, be1, W2, b2, g2, be2):
    x2 = x.reshape(_M, _FIN)
    ek = _edge_keys(faces)
    g, vals = _sc_topk_gather(ek, x2)
    g3 = g.reshape(_M, _K, _FIN)
    v3 = vals.reshape(_M, _K, 1)

    W1full = jnp.transpose(W1[:, :, 0, :], (2, 1, 0))
    W1a = W1full[0]
    W1b = W1full[1:]
    W2t = W2[:, :, 0].T
    return _dense_chain(x2, g3, v3, W1a, W1b, b1, g1, be1, W2t, b2, g2, be2)


# speedup vs baseline: 2.5671x; 1.0924x over previous
import functools
import jax
import jax.numpy as jnp
from jax import lax
from jax.experimental import pallas as pl
from jax.experimental.pallas import tpu as pltpu
from jax.experimental.pallas import tpu_sc as plsc

_B, _N, _NF = 8, 2048, 4096
_K, _FIN, _FOUT, _FOUT2 = 16, 128, 256, 256
_EPS = 1e-5
_M = _B * _N
_NE = 6 * _NF
_ROWS = 256
_STEPS = _M // _ROWS
_RG = 32


def _sc_topk_gather(edge_keys, x2):
    mesh = plsc.VectorSubcoreMesh(
        core_axis_name="c", subcore_axis_name="s", num_cores=2, num_subcores=16)

    @functools.partial(
        pl.kernel,
        out_type=[
            jax.ShapeDtypeStruct((_M * _K, _FIN), jnp.float32),
            jax.ShapeDtypeStruct((_M * _K,), jnp.float32),
        ],
        mesh=mesh,
        compiler_params=pltpu.CompilerParams(needs_layout_passes=False),
        scratch_types=[
            pltpu.VMEM((_RG * 2049,), jnp.float32),
            pltpu.VMEM((_NE,), jnp.int32),
            pltpu.VMEM((_RG * 17,), jnp.int32),
            pltpu.VMEM((512,), jnp.float32),
            pltpu.VMEM((512,), jnp.int32),
            pltpu.VMEM((128, _FIN), jnp.float32),
            pltpu.VMEM((128, _FIN), jnp.float32),
            pltpu.SemaphoreType.DMA,
            pltpu.SemaphoreType.DMA,
        ],
    )
    def body(ek_hbm, x_hbm, g_hbm, v_hbm, mat, ekeys, nbrbuf, valbuf, gidx,
             gbufa, gbufb, sema, semb):
        c = lax.axis_index("c")
        s = lax.axis_index("s")
        lanes = lax.iota(jnp.int32, 16)
        zero16 = jnp.zeros((16,), jnp.float32)

        def zrow(i, carry):
            mat[pl.ds(i * 16, 16)] = zero16
            return carry
        lax.fori_loop(0, _RG * 2049 // 16, zrow, 0, unroll=8)

        def task(j, carry):
            b = 4 * c + j // 4
            rg = s + 16 * (j % 4)
            r0 = rg * _RG

            @pl.when((j % 4) == 0)
            def _():
                pltpu.sync_copy(ek_hbm.at[pl.ds(b * _NE, _NE)], ekeys)

            def sed(i, carry):
                k = ekeys[pl.ds(i * 16, 16)]
                src = k >> 11
                dst = k & 2047
                m = (src >= r0) & (src < r0 + _RG)
                addr = jnp.where(m, (src - r0) * 2049 + dst, 0)
                val = jnp.where(src == dst, 2.0, 1.0).astype(jnp.float32)
                plsc.store_scatter(mat, [addr], val, mask=m)
                return carry
            lax.fori_loop(0, _NE // 16, sed, 0, unroll=4)

            for g in range(2):
                lrow = g * 16 + lanes
                prow = r0 + lrow
                abase = lrow * 2049
                nbase = lrow * 17
                dv = plsc.load_gather(mat, [abase + prow])

                def colstep(cc, cnt):
                    w = plsc.load_gather(mat, [abase + cc])
                    m = (w != 0.0) & (prow != cc)
                    wm = m & (cnt < 16)
                    plsc.store_scatter(
                        nbrbuf, [nbase + jnp.minimum(cnt, 15)],
                        jnp.full((16,), cc, jnp.int32), mask=wm)
                    return cnt + m.astype(jnp.int32)
                cnt = lax.fori_loop(0, _N, colstep,
                                    jnp.zeros((16,), jnp.int32), unroll=8)

                hd = (dv > 0.0).astype(jnp.int32)
                for t in range(_K):
                    tt = t - hd
                    nb = plsc.load_gather(nbrbuf, [nbase + jnp.maximum(tt, 0)])
                    one = (tt >= 0) & (tt < cnt)
                    if t == 0:
                        isd = hd == 1
                        vals_t = jnp.where(isd, 2.0,
                                           jnp.where(one, 1.0, 0.0))
                        idx_t = jnp.where(isd, prow, jnp.where(one, nb, 0))
                    else:
                        vals_t = jnp.where(one, 1.0, 0.0)
                        idx_t = jnp.where(one, nb, 0)
                    pos = lrow * 16 + t
                    plsc.store_scatter(valbuf, [pos], vals_t)
                    plsc.store_scatter(gidx, [pos], b * 2048 + idx_t)

            def used(i, carry):
                k = ekeys[pl.ds(i * 16, 16)]
                src = k >> 11
                dst = k & 2047
                m = (src >= r0) & (src < r0 + _RG)
                addr = jnp.where(m, (src - r0) * 2049 + dst, 0)
                plsc.store_scatter(mat, [addr], zero16, mask=m)
                return carry
            lax.fori_loop(0, _NE // 16, used, 0, unroll=4)

            out0 = (b * 2048 + r0) * _K
            bufs = (gbufa, gbufb)
            sems = (sema, semb)
            descs = [None, None]
            descs[0] = pltpu.async_copy(
                x_hbm.at[gidx.at[pl.ds(0, 128)]], bufs[0], sems[0])
            for q in range(4):
                if q < 3:
                    descs[(q + 1) % 2] = pltpu.async_copy(
                        x_hbm.at[gidx.at[pl.ds((q + 1) * 128, 128)]],
                        bufs[(q + 1) % 2], sems[(q + 1) % 2])
                descs[q % 2].wait()
                pltpu.sync_copy(bufs[q % 2],
                                g_hbm.at[pl.ds(out0 + q * 128, 128)])
            pltpu.sync_copy(valbuf, v_hbm.at[pl.ds(out0, 512)])
            return carry

        lax.fori_loop(0, 16, task, 0)

    return body(edge_keys, x2)


def _conv1_body(x_ref, g_ref, v_ref, wa_ref, wb_ref, b_ref,
                h_ref, stat_ref, acc_ref):
    i = pl.program_id(0)

    @pl.when(i == 0)
    def _():
        acc_ref[...] = jnp.zeros_like(acc_ref)

    w = g_ref[...] * v_ref[...]
    h = jnp.dot(x_ref[...], wa_ref[...], preferred_element_type=jnp.float32)
    for t in range(_K):
        h += jnp.dot(w[:, t, :], wb_ref[t], preferred_element_type=jnp.float32)
    h = h + b_ref[...]
    h_ref[...] = h
    acc_ref[0, :] += h.sum(axis=0)
    acc_ref[1, :] += (h * h).sum(axis=0)

    @pl.when(i == _STEPS - 1)
    def _():
        stat_ref[...] = acc_ref[...]


def _bn_pool_body(h_ref, sc_ref, sh_ref, p_ref, stat_ref, acc_ref):
    i = pl.program_id(0)

    @pl.when(i == 0)
    def _():
        acc_ref[...] = jnp.zeros_like(acc_ref)

    a = jnp.maximum(h_ref[...] * sc_ref[...] + sh_ref[...], 0.0)
    p = a.reshape(_ROWS, _FOUT // _K, _K).max(axis=-1)
    p_ref[...] = p
    acc_ref[0:16, :] += jnp.dot(p.T, p, preferred_element_type=jnp.float32)
    acc_ref[16, :] += p.sum(axis=0)

    @pl.when(i == _STEPS - 1)
    def _():
        stat_ref[...] = acc_ref[...]


def _conv2_body(p_ref, w_ref, sc_ref, sh_ref, o_ref):
    h = jnp.dot(p_ref[...], w_ref[...], preferred_element_type=jnp.float32)
    o_ref[...] = jnp.maximum(h * sc_ref[...] + sh_ref[...], 0.0)


def _dense_chain(x2, g3, v3, W1a, W1b, b1, g1, be1, W2t, b2, g2, be2):
    h1, st1 = pl.pallas_call(
        _conv1_body,
        grid=(_STEPS,),
        in_specs=[
            pl.BlockSpec((_ROWS, _FIN), lambda i: (i, 0)),
            pl.BlockSpec((_ROWS, _K, _FIN), lambda i: (i, 0, 0)),
            pl.BlockSpec((_ROWS, _K, 1), lambda i: (i, 0, 0)),
            pl.BlockSpec((_FIN, _FOUT), lambda i: (0, 0)),
            pl.BlockSpec((_K, _FIN, _FOUT), lambda i: (0, 0, 0)),
            pl.BlockSpec((1, _FOUT), lambda i: (0, 0)),
        ],
        out_specs=[
            pl.BlockSpec((_ROWS, _FOUT), lambda i: (i, 0)),
            pl.BlockSpec((2, _FOUT), lambda i: (0, 0)),
        ],
        out_shape=[
            jax.ShapeDtypeStruct((_M, _FOUT), jnp.float32),
            jax.ShapeDtypeStruct((2, _FOUT), jnp.float32),
        ],
        scratch_shapes=[pltpu.VMEM((2, _FOUT), jnp.float32)],
    )(x2, g3, v3, W1a, W1b, b1.reshape(1, _FOUT))

    mean1 = st1[0] / _M
    var1 = st1[1] / _M - mean1 * mean1
    sc1 = g1 / jnp.sqrt(var1 + _EPS)
    sh1 = be1 - mean1 * sc1

    pooled, st2 = pl.pallas_call(
        _bn_pool_body,
        grid=(_STEPS,),
        in_specs=[
            pl.BlockSpec((_ROWS, _FOUT), lambda i: (i, 0)),
            pl.BlockSpec((1, _FOUT), lambda i: (0, 0)),
            pl.BlockSpec((1, _FOUT), lambda i: (0, 0)),
        ],
        out_specs=[
            pl.BlockSpec((_ROWS, _FOUT // _K), lambda i: (i, 0)),
            pl.BlockSpec((17, _FOUT // _K), lambda i: (0, 0)),
        ],
        out_shape=[
            jax.ShapeDtypeStruct((_M, _FOUT // _K), jnp.float32),
            jax.ShapeDtypeStruct((17, _FOUT // _K), jnp.float32),
        ],
        scratch_shapes=[pltpu.VMEM((17, _FOUT // _K), jnp.float32)],
    )(h1, sc1.reshape(1, _FOUT), sh1.reshape(1, _FOUT))

    S = st2[0:16, :]
    meanp = st2[16, :] / _M
    cov = S / _M - meanp[:, None] * meanp[None, :]
    mean2 = W2t.T @ meanp + b2
    var2 = jnp.einsum("co,cd,do->o", W2t, cov, W2t)
    sc2 = g2 / jnp.sqrt(var2 + _EPS)
    sh2 = be2 - mean2 * sc2

    out = pl.pallas_call(
        _conv2_body,
        grid=(_STEPS,),
        in_specs=[
            pl.BlockSpec((_ROWS, _FOUT // _K), lambda i: (i, 0)),
            pl.BlockSpec((_FOUT // _K, _FOUT2), lambda i: (0, 0)),
            pl.BlockSpec((1, _FOUT2), lambda i: (0, 0)),
            pl.BlockSpec((1, _FOUT2), lambda i: (0, 0)),
        ],
        out_specs=pl.BlockSpec((_ROWS, _FOUT2), lambda i: (i, 0)),
        out_shape=jax.ShapeDtypeStruct((_M, _FOUT2), jnp.float32),
    )(pooled, W2t, sc2.reshape(1, _FOUT2), sh2.reshape(1, _FOUT2))
    return out.reshape(_B, _N, _FOUT2)


def _edge_keys(faces):
    f0, f1, f2 = faces[..., 0], faces[..., 1], faces[..., 2]
    mn = jnp.minimum(jnp.minimum(f0, f1), f2)
    mx = jnp.maximum(jnp.maximum(f0, f1), f2)
    md = f0 + f1 + f2 - mn - mx
    fi = jnp.arange(_NF, dtype=jnp.int32)
    first_bad = jnp.min(jnp.where(mn == md, fi, _NF), axis=-1)
    valid = fi[None, :] < first_bad[:, None]

    def enc(a, d):
        return jnp.where(valid, a * 2048 + d, -1)

    keys = jnp.stack(
        [enc(mn, md), enc(md, mn), enc(mn, mx),
         enc(mx, mn), enc(md, mx), enc(mx, md)], axis=1)
    return keys.reshape(_B * _NE).astype(jnp.int32)


def kernel(x, faces, W1, b1, g1, be1, W2, b2, g2, be2):
    x2 = x.reshape(_M, _FIN)
    ek = _edge_keys(faces)
    g, vals = _sc_topk_gather(ek, x2)
    g3 = g.reshape(_M, _K, _FIN)
    v3 = vals.reshape(_M, _K, 1)

    W1full = jnp.transpose(W1[:, :, 0, :], (2, 1, 0))
    W1a = W1full[0]
    W1b = W1full[1:]
    W2t = W2[:, :, 0].T
    return _dense_chain(x2, g3, v3, W1a, W1b, b1, g1, be1, W2t, b2, g2, be2)


# speedup vs baseline: 2.5699x; 1.0011x over previous
import functools
import jax
import jax.numpy as jnp
from jax import lax
from jax.experimental import pallas as pl
from jax.experimental.pallas import tpu as pltpu
from jax.experimental.pallas import tpu_sc as plsc

_B, _N, _NF = 8, 2048, 4096
_K, _FIN, _FOUT, _FOUT2 = 16, 128, 256, 256
_EPS = 1e-5
_M = _B * _N
_NE = 6 * _NF
_ROWS = 256
_STEPS = _M // _ROWS
_RG = 32


def _sc_topk_gather(edge_keys, x2):
    mesh = plsc.VectorSubcoreMesh(
        core_axis_name="c", subcore_axis_name="s", num_cores=2, num_subcores=16)

    @functools.partial(
        pl.kernel,
        out_type=[
            jax.ShapeDtypeStruct((_M * _K, _FIN), jnp.float32),
            jax.ShapeDtypeStruct((_M * _K,), jnp.float32),
        ],
        mesh=mesh,
        compiler_params=pltpu.CompilerParams(needs_layout_passes=False),
        scratch_types=[
            pltpu.VMEM((_RG * 2049,), jnp.float32),
            pltpu.VMEM((_NE,), jnp.int32),
            pltpu.VMEM((_RG * 17,), jnp.int32),
            pltpu.VMEM((512,), jnp.float32),
            pltpu.VMEM((512,), jnp.int32),
            pltpu.VMEM((128, _FIN), jnp.float32),
            pltpu.VMEM((128, _FIN), jnp.float32),
            pltpu.SemaphoreType.DMA,
            pltpu.SemaphoreType.DMA,
        ],
    )
    def body(ek_hbm, x_hbm, g_hbm, v_hbm, mat, ekeys, nbrbuf, valbuf, gidx,
             gbufa, gbufb, sema, semb):
        c = lax.axis_index("c")
        s = lax.axis_index("s")
        lanes = lax.iota(jnp.int32, 16)
        zero16 = jnp.zeros((16,), jnp.float32)

        def zrow(i, carry):
            mat[pl.ds(i * 16, 16)] = zero16
            return carry
        lax.fori_loop(0, _RG * 2049 // 16, zrow, 0, unroll=8)

        def task(j, carry):
            b = 4 * c + j // 4
            rg = s + 16 * (j % 4)
            r0 = rg * _RG

            @pl.when((j % 4) == 0)
            def _():
                pltpu.sync_copy(ek_hbm.at[pl.ds(b * _NE, _NE)], ekeys)

            def sed(i, carry):
                k = ekeys[pl.ds(i * 16, 16)]
                src = k >> 11
                dst = k & 2047
                m = (src >= r0) & (src < r0 + _RG)
                addr = jnp.where(m, (src - r0) * 2049 + dst, 0)
                val = jnp.where(src == dst, 2.0, 1.0).astype(jnp.float32)
                plsc.store_scatter(mat, [addr], val, mask=m)
                return carry
            with jax.named_scope("ph_scatter"):
                lax.fori_loop(0, _NE // 16, sed, 0, unroll=4)

            for g in range(2):
                lrow = g * 16 + lanes
                prow = r0 + lrow
                abase = lrow * 2049
                nbase = lrow * 17
                dv = plsc.load_gather(mat, [abase + prow])

                def colstep(cc, cnt):
                    w = plsc.load_gather(mat, [abase + cc])
                    m = (w != 0.0) & (prow != cc)
                    wm = m & (cnt < 16)
                    plsc.store_scatter(
                        nbrbuf, [nbase + jnp.minimum(cnt, 15)],
                        jnp.full((16,), cc, jnp.int32), mask=wm)
                    return cnt + m.astype(jnp.int32)
                with jax.named_scope("ph_scan"):
                    cnt = lax.fori_loop(0, _N, colstep,
                                        jnp.zeros((16,), jnp.int32), unroll=8)

                hd = (dv > 0.0).astype(jnp.int32)
                for t in range(_K):
                    tt = t - hd
                    nb = plsc.load_gather(nbrbuf, [nbase + jnp.maximum(tt, 0)])
                    one = (tt >= 0) & (tt < cnt)
                    if t == 0:
                        isd = hd == 1
                        vals_t = jnp.where(isd, 2.0,
                                           jnp.where(one, 1.0, 0.0))
                        idx_t = jnp.where(isd, prow, jnp.where(one, nb, 0))
                    else:
                        vals_t = jnp.where(one, 1.0, 0.0)
                        idx_t = jnp.where(one, nb, 0)
                    pos = lrow * 16 + t
                    plsc.store_scatter(valbuf, [pos], vals_t)
                    plsc.store_scatter(gidx, [pos], b * 2048 + idx_t)

            def used(i, carry):
                k = ekeys[pl.ds(i * 16, 16)]
                src = k >> 11
                dst = k & 2047
                m = (src >= r0) & (src < r0 + _RG)
                addr = jnp.where(m, (src - r0) * 2049 + dst, 0)
                plsc.store_scatter(mat, [addr], zero16, mask=m)
                return carry
            with jax.named_scope("ph_unscatter"):
                lax.fori_loop(0, _NE // 16, used, 0, unroll=4)

            with jax.named_scope("ph_dma"):
                out0 = (b * 2048 + r0) * _K
                bufs = (gbufa, gbufb)
                sems = (sema, semb)
                descs = [None, None]
                descs[0] = pltpu.async_copy(
                    x_hbm.at[gidx.at[pl.ds(0, 128)]], bufs[0], sems[0])
                for q in range(4):
                    if q < 3:
                        descs[(q + 1) % 2] = pltpu.async_copy(
                            x_hbm.at[gidx.at[pl.ds((q + 1) * 128, 128)]],
                            bufs[(q + 1) % 2], sems[(q + 1) % 2])
                    descs[q % 2].wait()
                    pltpu.sync_copy(bufs[q % 2],
                                    g_hbm.at[pl.ds(out0 + q * 128, 128)])
                pltpu.sync_copy(valbuf, v_hbm.at[pl.ds(out0, 512)])
            return carry

        lax.fori_loop(0, 16, task, 0)

    return body(edge_keys, x2)


def _conv1_body(x_ref, g_ref, v_ref, wa_ref, wb_ref, b_ref,
                h_ref, stat_ref, acc_ref):
    i = pl.program_id(0)

    @pl.when(i == 0)
    def _():
        acc_ref[...] = jnp.zeros_like(acc_ref)

    w = g_ref[...] * v_ref[...]
    h = jnp.dot(x_ref[...], wa_ref[...], preferred_element_type=jnp.float32)
    for t in range(_K):
        h += jnp.dot(w[:, t, :], wb_ref[t], preferred_element_type=jnp.float32)
    h = h + b_ref[...]
    h_ref[...] = h
    acc_ref[0, :] += h.sum(axis=0)
    acc_ref[1, :] += (h * h).sum(axis=0)

    @pl.when(i == _STEPS - 1)
    def _():
        stat_ref[...] = acc_ref[...]


def _bn_pool_body(h_ref, sc_ref, sh_ref, p_ref, stat_ref, acc_ref):
    i = pl.program_id(0)

    @pl.when(i == 0)
    def _():
        acc_ref[...] = jnp.zeros_like(acc_ref)

    a = jnp.maximum(h_ref[...] * sc_ref[...] + sh_ref[...], 0.0)
    p = a.reshape(_ROWS, _FOUT // _K, _K).max(axis=-1)
    p_ref[...] = p
    acc_ref[0:16, :] += jnp.dot(p.T, p, preferred_element_type=jnp.float32)
    acc_ref[16, :] += p.sum(axis=0)

    @pl.when(i == _STEPS - 1)
    def _():
        stat_ref[...] = acc_ref[...]


def _conv2_body(p_ref, w_ref, sc_ref, sh_ref, o_ref):
    h = jnp.dot(p_ref[...], w_ref[...], preferred_element_type=jnp.float32)
    o_ref[...] = jnp.maximum(h * sc_ref[...] + sh_ref[...], 0.0)


def _dense_chain(x2, g3, v3, W1a, W1b, b1, g1, be1, W2t, b2, g2, be2):
    h1, st1 = pl.pallas_call(
        _conv1_body,
        grid=(_STEPS,),
        in_specs=[
            pl.BlockSpec((_ROWS, _FIN), lambda i: (i, 0)),
            pl.BlockSpec((_ROWS, _K, _FIN), lambda i: (i, 0, 0)),
            pl.BlockSpec((_ROWS, _K, 1), lambda i: (i, 0, 0)),
            pl.BlockSpec((_FIN, _FOUT), lambda i: (0, 0)),
            pl.BlockSpec((_K, _FIN, _FOUT), lambda i: (0, 0, 0)),
            pl.BlockSpec((1, _FOUT), lambda i: (0, 0)),
        ],
        out_specs=[
            pl.BlockSpec((_ROWS, _FOUT), lambda i: (i, 0)),
            pl.BlockSpec((2, _FOUT), lambda i: (0, 0)),
        ],
        out_shape=[
            jax.ShapeDtypeStruct((_M, _FOUT), jnp.float32),
            jax.ShapeDtypeStruct((2, _FOUT), jnp.float32),
        ],
        scratch_shapes=[pltpu.VMEM((2, _FOUT), jnp.float32)],
    )(x2, g3, v3, W1a, W1b, b1.reshape(1, _FOUT))

    mean1 = st1[0] / _M
    var1 = st1[1] / _M - mean1 * mean1
    sc1 = g1 / jnp.sqrt(var1 + _EPS)
    sh1 = be1 - mean1 * sc1

    pooled, st2 = pl.pallas_call(
        _bn_pool_body,
        grid=(_STEPS,),
        in_specs=[
            pl.BlockSpec((_ROWS, _FOUT), lambda i: (i, 0)),
            pl.BlockSpec((1, _FOUT), lambda i: (0, 0)),
            pl.BlockSpec((1, _FOUT), lambda i: (0, 0)),
        ],
        out_specs=[
            pl.BlockSpec((_ROWS, _FOUT // _K), lambda i: (i, 0)),
            pl.BlockSpec((17, _FOUT // _K), lambda i: (0, 0)),
        ],
        out_shape=[
            jax.ShapeDtypeStruct((_M, _FOUT // _K), jnp.float32),
            jax.ShapeDtypeStruct((17, _FOUT // _K), jnp.float32),
        ],
        scratch_shapes=[pltpu.VMEM((17, _FOUT // _K), jnp.float32)],
    )(h1, sc1.reshape(1, _FOUT), sh1.reshape(1, _FOUT))

    S = st2[0:16, :]
    meanp = st2[16, :] / _M
    cov = S / _M - meanp[:, None] * meanp[None, :]
    mean2 = W2t.T @ meanp + b2
    var2 = jnp.einsum("co,cd,do->o", W2t, cov, W2t)
    sc2 = g2 / jnp.sqrt(var2 + _EPS)
    sh2 = be2 - mean2 * sc2

    out = pl.pallas_call(
        _conv2_body,
        grid=(_STEPS,),
        in_specs=[
            pl.BlockSpec((_ROWS, _FOUT // _K), lambda i: (i, 0)),
            pl.BlockSpec((_FOUT // _K, _FOUT2), lambda i: (0, 0)),
            pl.BlockSpec((1, _FOUT2), lambda i: (0, 0)),
            pl.BlockSpec((1, _FOUT2), lambda i: (0, 0)),
        ],
        out_specs=pl.BlockSpec((_ROWS, _FOUT2), lambda i: (i, 0)),
        out_shape=jax.ShapeDtypeStruct((_M, _FOUT2), jnp.float32),
    )(pooled, W2t, sc2.reshape(1, _FOUT2), sh2.reshape(1, _FOUT2))
    return out.reshape(_B, _N, _FOUT2)


def _edge_keys(faces):
    f0, f1, f2 = faces[..., 0], faces[..., 1], faces[..., 2]
    mn = jnp.minimum(jnp.minimum(f0, f1), f2)
    mx = jnp.maximum(jnp.maximum(f0, f1), f2)
    md = f0 + f1 + f2 - mn - mx
    fi = jnp.arange(_NF, dtype=jnp.int32)
    first_bad = jnp.min(jnp.where(mn == md, fi, _NF), axis=-1)
    valid = fi[None, :] < first_bad[:, None]

    def enc(a, d):
        return jnp.where(valid, a * 2048 + d, -1)

    keys = jnp.stack(
        [enc(mn, md), enc(md, mn), enc(mn, mx),
         enc(mx, mn), enc(md, mx), enc(mx, md)], axis=1)
    return keys.reshape(_B * _NE).astype(jnp.int32)


def kernel(x, faces, W1, b1, g1, be1, W2, b2, g2, be2):
    x2 = x.reshape(_M, _FIN)
    ek = _edge_keys(faces)
    g, vals = _sc_topk_gather(ek, x2)
    g3 = g.reshape(_M, _K, _FIN)
    v3 = vals.reshape(_M, _K, 1)

    W1full = jnp.transpose(W1[:, :, 0, :], (2, 1, 0))
    W1a = W1full[0]
    W1b = W1full[1:]
    W2t = W2[:, :, 0].T
    return _dense_chain(x2, g3, v3, W1a, W1b, b1, g1, be1, W2t, b2, g2, be2)


# speedup vs baseline: 16.0386x; 6.2410x over previous
import functools
import jax
import jax.numpy as jnp
from jax import lax
from jax.experimental import pallas as pl
from jax.experimental.pallas import tpu as pltpu
from jax.experimental.pallas import tpu_sc as plsc

_B, _N, _NF = 8, 2048, 4096
_K, _FIN, _FOUT, _FOUT2 = 16, 128, 256, 256
_EPS = 1e-5
_M = _B * _N
_NE = 6 * _NF
_ROWS = 256
_STEPS = _M // _ROWS
_RG = 32


def _sc_topk_gather(edge_keys, x2):
    mesh = plsc.VectorSubcoreMesh(
        core_axis_name="c", subcore_axis_name="s", num_cores=2, num_subcores=16)

    @functools.partial(
        pl.kernel,
        out_type=[
            jax.ShapeDtypeStruct((_M * _K, _FIN), jnp.float32),
            jax.ShapeDtypeStruct((_M * _K,), jnp.float32),
        ],
        mesh=mesh,
        compiler_params=pltpu.CompilerParams(needs_layout_passes=False),
        scratch_types=[
            pltpu.VMEM((_N * _RG,), jnp.float32),
            pltpu.VMEM((_NE,), jnp.int32),
            pltpu.VMEM((_RG * 17,), jnp.int32),
            pltpu.VMEM((_RG,), jnp.float32),
            pltpu.VMEM((512,), jnp.float32),
            pltpu.VMEM((512,), jnp.int32),
            pltpu.VMEM((64, _FIN), jnp.float32),
            pltpu.VMEM((64, _FIN), jnp.float32),
            pltpu.VMEM_SHARED((_N, _FIN), jnp.float32),
            pltpu.SemaphoreType.DMA,
            pltpu.SemaphoreType.DMA,
        ],
    )
    def body(ek_hbm, x_hbm, g_hbm, v_hbm, mat, ekeys, nbrbuf, diagf, valbuf,
             gidx, gbufa, gbufb, xs, sema, semb):
        c = lax.axis_index("c")
        s = lax.axis_index("s")
        lanes = lax.iota(jnp.int32, 16)
        zero16 = jnp.zeros((16,), jnp.float32)
        one16 = jnp.ones((16,), jnp.float32)

        @plsc.parallel_loop(0, _N * _RG, 16, unroll=8)
        def _(i):
            mat[pl.ds(i, 16)] = zero16
        diagf[pl.ds(0, 16)] = zero16
        diagf[pl.ds(16, 16)] = zero16

        def task(j, carry):
            b = 4 * c + j // 4
            rg = s + 16 * (j % 4)
            r0 = rg * _RG

            @pl.when((j % 4) == 0)
            def _():
                plsc.subcore_barrier()
                pltpu.sync_copy(ek_hbm.at[pl.ds(b * _NE, _NE)], ekeys)
                pltpu.sync_copy(x_hbm.at[pl.ds(b * _N + s * 128, 128)],
                                xs.at[pl.ds(s * 128, 128)])
                plsc.subcore_barrier()

            with jax.named_scope("ph_scatter"):
                @plsc.parallel_loop(0, _NE // 16, 1, unroll=4)
                def _(i):
                    k = ekeys[pl.ds(i * 16, 16)]
                    src = k >> 11
                    dst = k & 2047
                    m = (src >= r0) & (src < r0 + _RG)
                    isd = src == dst
                    addr = jnp.where(m, dst * _RG + (src - r0), 0)
                    val = jnp.where(isd, 0.0, 1.0).astype(jnp.float32)
                    plsc.store_scatter(mat, [addr], val, mask=m)
                    plsc.store_scatter(diagf, [jnp.where(m, src - r0, 0)],
                                       one16, mask=m & isd)

            for g in range(2):
                lrow = g * 16 + lanes
                prow = r0 + lrow
                nbase = lrow * 17
                dv = diagf[pl.ds(g * 16, 16)]

                with jax.named_scope("ph_scan"):
                    @plsc.parallel_loop(
                        0, _N, 1, unroll=8,
                        carry=jnp.zeros((16,), jnp.int32))
                    def cnt(cc, cnt):
                        w = mat[pl.ds(cc * _RG + g * 16, 16)]
                        m = w != 0.0
                        plsc.store_scatter(
                            nbrbuf, [nbase + jnp.minimum(cnt, 16)],
                            jnp.full((16,), cc, jnp.int32), mask=m)
                        return cnt + m.astype(jnp.int32)

                hd = (dv > 0.0).astype(jnp.int32)
                for t in range(_K):
                    tt = t - hd
                    nb = plsc.load_gather(nbrbuf, [nbase + jnp.maximum(tt, 0)])
                    one = (tt >= 0) & (tt < cnt)
                    if t == 0:
                        isd = hd == 1
                        vals_t = jnp.where(isd, 2.0,
                                           jnp.where(one, 1.0, 0.0))
                        idx_t = jnp.where(isd, prow, jnp.where(one, nb, 0))
                    else:
                        vals_t = jnp.where(one, 1.0, 0.0)
                        idx_t = jnp.where(one, nb, 0)
                    pos = lrow * 16 + t
                    plsc.store_scatter(valbuf, [pos], vals_t)
                    plsc.store_scatter(gidx, [pos], idx_t)

            with jax.named_scope("ph_unscatter"):
                @plsc.parallel_loop(0, _NE // 16, 1, unroll=4)
                def _(i):
                    k = ekeys[pl.ds(i * 16, 16)]
                    src = k >> 11
                    dst = k & 2047
                    m = (src >= r0) & (src < r0 + _RG)
                    addr = jnp.where(m, dst * _RG + (src - r0), 0)
                    plsc.store_scatter(mat, [addr], zero16, mask=m)
                diagf[pl.ds(0, 16)] = zero16
                diagf[pl.ds(16, 16)] = zero16

            with jax.named_scope("ph_dma"):
                out0 = (b * 2048 + r0) * _K
                bufs = (gbufa, gbufb)
                sems = (sema, semb)
                descs = [None, None]
                descs[0] = pltpu.async_copy(
                    xs.at[gidx.at[pl.ds(0, 64)]], bufs[0], sems[0])
                for q in range(8):
                    if q < 7:
                        descs[(q + 1) % 2] = pltpu.async_copy(
                            xs.at[gidx.at[pl.ds((q + 1) * 64, 64)]],
                            bufs[(q + 1) % 2], sems[(q + 1) % 2])
                    descs[q % 2].wait()
                    pltpu.sync_copy(bufs[q % 2],
                                    g_hbm.at[pl.ds(out0 + q * 64, 64)])
                pltpu.sync_copy(valbuf, v_hbm.at[pl.ds(out0, 512)])
            return carry

        lax.fori_loop(0, 16, task, 0)

    return body(edge_keys, x2)


def _conv1_body(x_ref, g_ref, v_ref, wa_ref, wb_ref, b_ref,
                h_ref, stat_ref, acc_ref):
    i = pl.program_id(0)

    @pl.when(i == 0)
    def _():
        acc_ref[...] = jnp.zeros_like(acc_ref)

    w = g_ref[...] * v_ref[...]
    h = jnp.dot(x_ref[...], wa_ref[...], preferred_element_type=jnp.float32)
    for t in range(_K):
        h += jnp.dot(w[:, t, :], wb_ref[t], preferred_element_type=jnp.float32)
    h = h + b_ref[...]
    h_ref[...] = h
    acc_ref[0, :] += h.sum(axis=0)
    acc_ref[1, :] += (h * h).sum(axis=0)

    @pl.when(i == _STEPS - 1)
    def _():
        stat_ref[...] = acc_ref[...]


def _bn_pool_body(h_ref, sc_ref, sh_ref, p_ref, stat_ref, acc_ref):
    i = pl.program_id(0)

    @pl.when(i == 0)
    def _():
        acc_ref[...] = jnp.zeros_like(acc_ref)

    a = jnp.maximum(h_ref[...] * sc_ref[...] + sh_ref[...], 0.0)
    p = a.reshape(_ROWS, _FOUT // _K, _K).max(axis=-1)
    p_ref[...] = p
    acc_ref[0:16, :] += jnp.dot(p.T, p, preferred_element_type=jnp.float32)
    acc_ref[16, :] += p.sum(axis=0)

    @pl.when(i == _STEPS - 1)
    def _():
        stat_ref[...] = acc_ref[...]


def _conv2_body(p_ref, w_ref, sc_ref, sh_ref, o_ref):
    h = jnp.dot(p_ref[...], w_ref[...], preferred_element_type=jnp.float32)
    o_ref[...] = jnp.maximum(h * sc_ref[...] + sh_ref[...], 0.0)


def _dense_chain(x2, g3, v3, W1a, W1b, b1, g1, be1, W2t, b2, g2, be2):
    h1, st1 = pl.pallas_call(
        _conv1_body,
        grid=(_STEPS,),
        in_specs=[
            pl.BlockSpec((_ROWS, _FIN), lambda i: (i, 0)),
            pl.BlockSpec((_ROWS, _K, _FIN), lambda i: (i, 0, 0)),
            pl.BlockSpec((_ROWS, _K, 1), lambda i: (i, 0, 0)),
            pl.BlockSpec((_FIN, _FOUT), lambda i: (0, 0)),
            pl.BlockSpec((_K, _FIN, _FOUT), lambda i: (0, 0, 0)),
            pl.BlockSpec((1, _FOUT), lambda i: (0, 0)),
        ],
        out_specs=[
            pl.BlockSpec((_ROWS, _FOUT), lambda i: (i, 0)),
            pl.BlockSpec((2, _FOUT), lambda i: (0, 0)),
        ],
        out_shape=[
            jax.ShapeDtypeStruct((_M, _FOUT), jnp.float32),
            jax.ShapeDtypeStruct((2, _FOUT), jnp.float32),
        ],
        scratch_shapes=[pltpu.VMEM((2, _FOUT), jnp.float32)],
    )(x2, g3, v3, W1a, W1b, b1.reshape(1, _FOUT))

    mean1 = st1[0] / _M
    var1 = st1[1] / _M - mean1 * mean1
    sc1 = g1 / jnp.sqrt(var1 + _EPS)
    sh1 = be1 - mean1 * sc1

    pooled, st2 = pl.pallas_call(
        _bn_pool_body,
        grid=(_STEPS,),
        in_specs=[
            pl.BlockSpec((_ROWS, _FOUT), lambda i: (i, 0)),
            pl.BlockSpec((1, _FOUT), lambda i: (0, 0)),
            pl.BlockSpec((1, _FOUT), lambda i: (0, 0)),
        ],
        out_specs=[
            pl.BlockSpec((_ROWS, _FOUT // _K), lambda i: (i, 0)),
            pl.BlockSpec((17, _FOUT // _K), lambda i: (0, 0)),
        ],
        out_shape=[
            jax.ShapeDtypeStruct((_M, _FOUT // _K), jnp.float32),
            jax.ShapeDtypeStruct((17, _FOUT // _K), jnp.float32),
        ],
        scratch_shapes=[pltpu.VMEM((17, _FOUT // _K), jnp.float32)],
    )(h1, sc1.reshape(1, _FOUT), sh1.reshape(1, _FOUT))

    S = st2[0:16, :]
    meanp = st2[16, :] / _M
    cov = S / _M - meanp[:, None] * meanp[None, :]
    mean2 = W2t.T @ meanp + b2
    var2 = jnp.einsum("co,cd,do->o", W2t, cov, W2t)
    sc2 = g2 / jnp.sqrt(var2 + _EPS)
    sh2 = be2 - mean2 * sc2

    out = pl.pallas_call(
        _conv2_body,
        grid=(_STEPS,),
        in_specs=[
            pl.BlockSpec((_ROWS, _FOUT // _K), lambda i: (i, 0)),
            pl.BlockSpec((_FOUT // _K, _FOUT2), lambda i: (0, 0)),
            pl.BlockSpec((1, _FOUT2), lambda i: (0, 0)),
            pl.BlockSpec((1, _FOUT2), lambda i: (0, 0)),
        ],
        out_specs=pl.BlockSpec((_ROWS, _FOUT2), lambda i: (i, 0)),
        out_shape=jax.ShapeDtypeStruct((_M, _FOUT2), jnp.float32),
    )(pooled, W2t, sc2.reshape(1, _FOUT2), sh2.reshape(1, _FOUT2))
    return out.reshape(_B, _N, _FOUT2)


def _edge_keys(faces):
    f0, f1, f2 = faces[..., 0], faces[..., 1], faces[..., 2]
    mn = jnp.minimum(jnp.minimum(f0, f1), f2)
    mx = jnp.maximum(jnp.maximum(f0, f1), f2)
    md = f0 + f1 + f2 - mn - mx
    fi = jnp.arange(_NF, dtype=jnp.int32)
    first_bad = jnp.min(jnp.where(mn == md, fi, _NF), axis=-1)
    valid = fi[None, :] < first_bad[:, None]

    def enc(a, d):
        return jnp.where(valid, a * 2048 + d, -1)

    keys = jnp.stack(
        [enc(mn, md), enc(md, mn), enc(mn, mx),
         enc(mx, mn), enc(md, mx), enc(mx, md)], axis=1)
    return keys.reshape(_B * _NE).astype(jnp.int32)


def kernel(x, faces, W1, b1, g1, be1, W2, b2, g2, be2):
    x2 = x.reshape(_M, _FIN)
    ek = _edge_keys(faces)
    g, vals = _sc_topk_gather(ek, x2)
    g3 = g.reshape(_M, _K, _FIN)
    v3 = vals.reshape(_M, _K, 1)

    W1full = jnp.transpose(W1[:, :, 0, :], (2, 1, 0))
    W1a = W1full[0]
    W1b = W1full[1:]
    W2t = W2[:, :, 0].T
    return _dense_chain(x2, g3, v3, W1a, W1b, b1, g1, be1, W2t, b2, g2, be2)


# speedup vs baseline: 20.5469x; 1.2811x over previous
import functools
import jax
import jax.numpy as jnp
from jax import lax
from jax.experimental import pallas as pl
from jax.experimental.pallas import tpu as pltpu
from jax.experimental.pallas import tpu_sc as plsc

_B, _N, _NF = 8, 2048, 4096
_K, _FIN, _FOUT, _FOUT2 = 16, 128, 256, 256
_EPS = 1e-5
_M = _B * _N
_NE = 6 * _NF
_ROWS = 256
_STEPS = _M // _ROWS
_RG = 32


def _sc_topk_gather(edge_keys, x2):
    mesh = plsc.VectorSubcoreMesh(
        core_axis_name="c", subcore_axis_name="s", num_cores=2, num_subcores=16)

    @functools.partial(
        pl.kernel,
        out_type=[
            jax.ShapeDtypeStruct((_K * _M, _FIN), jnp.float32),
            jax.ShapeDtypeStruct((_M,), jnp.float32),
        ],
        mesh=mesh,
        compiler_params=pltpu.CompilerParams(needs_layout_passes=False),
        scratch_types=[
            pltpu.VMEM((_N * _RG,), jnp.float32),
            pltpu.VMEM((_NE,), jnp.int32),
            pltpu.VMEM((_RG * 17,), jnp.int32),
            pltpu.VMEM((_RG,), jnp.float32),
            pltpu.VMEM((_RG,), jnp.float32),
            pltpu.VMEM((512,), jnp.int32),
            pltpu.VMEM((_RG, _FIN), jnp.float32),
            pltpu.VMEM((_RG, _FIN), jnp.float32),
            pltpu.VMEM_SHARED((_N + 1, _FIN), jnp.float32),
            pltpu.SemaphoreType.DMA,
            pltpu.SemaphoreType.DMA,
        ],
    )
    def body(ek_hbm, x_hbm, g_hbm, v_hbm, mat, ekeys, nbrbuf, diagf, valbuf,
             gidx, gbufa, gbufb, xs, sema, semb):
        c = lax.axis_index("c")
        s = lax.axis_index("s")
        lanes = lax.iota(jnp.int32, 16)
        zero16 = jnp.zeros((16,), jnp.float32)
        one16 = jnp.ones((16,), jnp.float32)

        @plsc.parallel_loop(0, _N * _RG, 16, unroll=8)
        def _(i):
            mat[pl.ds(i, 16)] = zero16
        diagf[pl.ds(0, 16)] = zero16
        diagf[pl.ds(16, 16)] = zero16
        for kk in range(_FIN // 16):
            gbufa[0, pl.ds(kk * 16, 16)] = zero16
        pltpu.sync_copy(gbufa.at[pl.ds(0, 1)], xs.at[pl.ds(_N, 1)])

        def task(j, carry):
            b = 4 * c + j // 4
            rg = s + 16 * (j % 4)
            r0 = rg * _RG

            @pl.when((j % 4) == 0)
            def _():
                plsc.subcore_barrier()
                pltpu.sync_copy(ek_hbm.at[pl.ds(b * _NE, _NE)], ekeys)
                pltpu.sync_copy(x_hbm.at[pl.ds(b * _N + s * 128, 128)],
                                xs.at[pl.ds(s * 128, 128)])
                plsc.subcore_barrier()

            with jax.named_scope("ph_scatter"):
                @plsc.parallel_loop(0, _NE // 16, 1, unroll=4)
                def _(i):
                    k = ekeys[pl.ds(i * 16, 16)]
                    src = k >> 11
                    dst = k & 2047
                    m = (src >= r0) & (src < r0 + _RG)
                    isd = src == dst
                    addr = jnp.where(m, dst * _RG + (src - r0), 0)
                    val = jnp.where(isd, 0.0, 1.0).astype(jnp.float32)
                    plsc.store_scatter(mat, [addr], val, mask=m)
                    plsc.store_scatter(diagf, [jnp.where(m, src - r0, 0)],
                                       one16, mask=m & isd)

            for g in range(2):
                lrow = g * 16 + lanes
                prow = r0 + lrow
                nbase = lrow * 17
                dv = diagf[pl.ds(g * 16, 16)]

                with jax.named_scope("ph_scan"):
                    @plsc.parallel_loop(
                        0, _N, 1, unroll=8,
                        carry=jnp.zeros((16,), jnp.int32))
                    def cnt(cc, cnt):
                        w = mat[pl.ds(cc * _RG + g * 16, 16)]
                        m = w != 0.0
                        plsc.store_scatter(
                            nbrbuf, [nbase + jnp.minimum(cnt, 16)],
                            jnp.full((16,), cc, jnp.int32), mask=m)
                        return cnt + m.astype(jnp.int32)

                hd = (dv > 0.0).astype(jnp.int32)
                for t in range(_K):
                    tt = t - hd
                    nb = plsc.load_gather(nbrbuf, [nbase + jnp.maximum(tt, 0)])
                    one = (tt >= 0) & (tt < cnt)
                    if t == 0:
                        isd = hd == 1
                        vals_t = jnp.where(isd, 2.0,
                                           jnp.where(one, 1.0, 0.0))
                        idx_t = jnp.where(isd, prow, jnp.where(one, nb, _N))
                        plsc.store_scatter(valbuf, [lrow], vals_t)
                    else:
                        idx_t = jnp.where(one, nb, _N)
                    plsc.store_scatter(gidx, [t * _RG + lrow], idx_t)

            with jax.named_scope("ph_unscatter"):
                @plsc.parallel_loop(0, _NE // 16, 1, unroll=4)
                def _(i):
                    k = ekeys[pl.ds(i * 16, 16)]
                    src = k >> 11
                    dst = k & 2047
                    m = (src >= r0) & (src < r0 + _RG)
                    addr = jnp.where(m, dst * _RG + (src - r0), 0)
                    plsc.store_scatter(mat, [addr], zero16, mask=m)
                diagf[pl.ds(0, 16)] = zero16
                diagf[pl.ds(16, 16)] = zero16

            with jax.named_scope("ph_dma"):
                out0 = b * 2048 + r0
                bufs = (gbufa, gbufb)
                sems = (sema, semb)
                descs = [None, None]
                descs[0] = pltpu.async_copy(
                    xs.at[gidx.at[pl.ds(0, _RG)]], bufs[0], sems[0])
                for q in range(_K):
                    if q < _K - 1:
                        descs[(q + 1) % 2] = pltpu.async_copy(
                            xs.at[gidx.at[pl.ds((q + 1) * _RG, _RG)]],
                            bufs[(q + 1) % 2], sems[(q + 1) % 2])
                    descs[q % 2].wait()
                    pltpu.sync_copy(bufs[q % 2],
                                    g_hbm.at[pl.ds(q * _M + out0, _RG)])
                pltpu.sync_copy(valbuf, v_hbm.at[pl.ds(out0, _RG)])
            return carry

        lax.fori_loop(0, 16, task, 0)

    return body(edge_keys, x2)


def _conv1_body(x_ref, g_ref, v_ref, wa_ref, wb_ref, b_ref,
                h_ref, stat_ref, acc_ref):
    i = pl.program_id(0)

    @pl.when(i == 0)
    def _():
        acc_ref[...] = jnp.zeros_like(acc_ref)

    h = jnp.dot(x_ref[...], wa_ref[...], preferred_element_type=jnp.float32)
    h += jnp.dot(g_ref[0] * v_ref[...], wb_ref[0],
                 preferred_element_type=jnp.float32)
    for t in range(1, _K):
        h += jnp.dot(g_ref[t], wb_ref[t], preferred_element_type=jnp.float32)
    h = h + b_ref[...]
    h_ref[...] = h
    acc_ref[0, :] += h.sum(axis=0)
    acc_ref[1, :] += (h * h).sum(axis=0)

    @pl.when(i == _STEPS - 1)
    def _():
        stat_ref[...] = acc_ref[...]


def _bn_pool_body(h_ref, sc_ref, sh_ref, p_ref, stat_ref, acc_ref):
    i = pl.program_id(0)

    @pl.when(i == 0)
    def _():
        acc_ref[...] = jnp.zeros_like(acc_ref)

    a = jnp.maximum(h_ref[...] * sc_ref[...] + sh_ref[...], 0.0)
    p = a.reshape(_ROWS, _FOUT // _K, _K).max(axis=-1)
    p_ref[...] = p
    acc_ref[0:16, :] += jnp.dot(p.T, p, preferred_element_type=jnp.float32)
    acc_ref[16, :] += p.sum(axis=0)

    @pl.when(i == _STEPS - 1)
    def _():
        stat_ref[...] = acc_ref[...]


def _conv2_body(p_ref, w_ref, sc_ref, sh_ref, o_ref):
    h = jnp.dot(p_ref[...], w_ref[...], preferred_element_type=jnp.float32)
    o_ref[...] = jnp.maximum(h * sc_ref[...] + sh_ref[...], 0.0)


def _dense_chain(x2, g3, v3, W1a, W1b, b1, g1, be1, W2t, b2, g2, be2):
    h1, st1 = pl.pallas_call(
        _conv1_body,
        grid=(_STEPS,),
        in_specs=[
            pl.BlockSpec((_ROWS, _FIN), lambda i: (i, 0)),
            pl.BlockSpec((_K, _ROWS, _FIN), lambda i: (0, i, 0)),
            pl.BlockSpec((_ROWS, 1), lambda i: (i, 0)),
            pl.BlockSpec((_FIN, _FOUT), lambda i: (0, 0)),
            pl.BlockSpec((_K, _FIN, _FOUT), lambda i: (0, 0, 0)),
            pl.BlockSpec((1, _FOUT), lambda i: (0, 0)),
        ],
        out_specs=[
            pl.BlockSpec((_ROWS, _FOUT), lambda i: (i, 0)),
            pl.BlockSpec((2, _FOUT), lambda i: (0, 0)),
        ],
        out_shape=[
            jax.ShapeDtypeStruct((_M, _FOUT), jnp.float32),
            jax.ShapeDtypeStruct((2, _FOUT), jnp.float32),
        ],
        scratch_shapes=[pltpu.VMEM((2, _FOUT), jnp.float32)],
    )(x2, g3, v3, W1a, W1b, b1.reshape(1, _FOUT))

    mean1 = st1[0] / _M
    var1 = st1[1] / _M - mean1 * mean1
    sc1 = g1 / jnp.sqrt(var1 + _EPS)
    sh1 = be1 - mean1 * sc1

    pooled, st2 = pl.pallas_call(
        _bn_pool_body,
        grid=(_STEPS,),
        in_specs=[
            pl.BlockSpec((_ROWS, _FOUT), lambda i: (i, 0)),
            pl.BlockSpec((1, _FOUT), lambda i: (0, 0)),
            pl.BlockSpec((1, _FOUT), lambda i: (0, 0)),
        ],
        out_specs=[
            pl.BlockSpec((_ROWS, _FOUT // _K), lambda i: (i, 0)),
            pl.BlockSpec((17, _FOUT // _K), lambda i: (0, 0)),
        ],
        out_shape=[
            jax.ShapeDtypeStruct((_M, _FOUT // _K), jnp.float32),
            jax.ShapeDtypeStruct((17, _FOUT // _K), jnp.float32),
        ],
        scratch_shapes=[pltpu.VMEM((17, _FOUT // _K), jnp.float32)],
    )(h1, sc1.reshape(1, _FOUT), sh1.reshape(1, _FOUT))

    S = st2[0:16, :]
    meanp = st2[16, :] / _M
    cov = S / _M - meanp[:, None] * meanp[None, :]
    mean2 = W2t.T @ meanp + b2
    var2 = jnp.einsum("co,cd,do->o", W2t, cov, W2t)
    sc2 = g2 / jnp.sqrt(var2 + _EPS)
    sh2 = be2 - mean2 * sc2

    out = pl.pallas_call(
        _conv2_body,
        grid=(_STEPS,),
        in_specs=[
            pl.BlockSpec((_ROWS, _FOUT // _K), lambda i: (i, 0)),
            pl.BlockSpec((_FOUT // _K, _FOUT2), lambda i: (0, 0)),
            pl.BlockSpec((1, _FOUT2), lambda i: (0, 0)),
            pl.BlockSpec((1, _FOUT2), lambda i: (0, 0)),
        ],
        out_specs=pl.BlockSpec((_ROWS, _FOUT2), lambda i: (i, 0)),
        out_shape=jax.ShapeDtypeStruct((_M, _FOUT2), jnp.float32),
    )(pooled, W2t, sc2.reshape(1, _FOUT2), sh2.reshape(1, _FOUT2))
    return out.reshape(_B, _N, _FOUT2)


def _edge_keys(faces):
    f0, f1, f2 = faces[..., 0], faces[..., 1], faces[..., 2]
    mn = jnp.minimum(jnp.minimum(f0, f1), f2)
    mx = jnp.maximum(jnp.maximum(f0, f1), f2)
    md = f0 + f1 + f2 - mn - mx
    fi = jnp.arange(_NF, dtype=jnp.int32)
    first_bad = jnp.min(jnp.where(mn == md, fi, _NF), axis=-1)
    valid = fi[None, :] < first_bad[:, None]

    def enc(a, d):
        return jnp.where(valid, a * 2048 + d, -1)

    keys = jnp.stack(
        [enc(mn, md), enc(md, mn), enc(mn, mx),
         enc(mx, mn), enc(md, mx), enc(mx, md)], axis=1)
    return keys.reshape(_B * _NE).astype(jnp.int32)


def kernel(x, faces, W1, b1, g1, be1, W2, b2, g2, be2):
    x2 = x.reshape(_M, _FIN)
    ek = _edge_keys(faces)
    g, vals = _sc_topk_gather(ek, x2)
    g3 = g.reshape(_K, _M, _FIN)
    v3 = vals.reshape(_M, 1)

    W1full = jnp.transpose(W1[:, :, 0, :], (2, 1, 0))
    W1a = W1full[0]
    W1b = W1full[1:]
    W2t = W2[:, :, 0].T
    return _dense_chain(x2, g3, v3, W1a, W1b, b1, g1, be1, W2t, b2, g2, be2)
